# baseline (speedup 1.0000x reference)
import jax
from jax.experimental import pallas as pl
from jax.experimental.pallas import tpu as pltpu


_N_CHUNKS = 16
_ROWS = 100000
_CHUNK = _ROWS // _N_CHUNKS


def _copy_body(emb_in, masks_in, emb_out, masks_out, sems, sem_m):
    copies = []
    for i in range(_N_CHUNKS):
        sl = pl.ds(i * _CHUNK, _CHUNK)
        c = pltpu.make_async_copy(emb_in.at[sl], emb_out.at[sl], sems.at[i])
        c.start()
        copies.append(c)
    cm = pltpu.make_async_copy(masks_in, masks_out, sem_m)
    cm.start()
    for c in copies:
        c.wait()
    cm.wait()


def kernel(embeddings, blade_masks):
    emb_out, masks_out = pl.pallas_call(
        _copy_body,
        in_specs=[
            pl.BlockSpec(memory_space=pl.ANY),
            pl.BlockSpec(memory_space=pl.ANY),
        ],
        out_specs=[
            pl.BlockSpec(memory_space=pl.ANY),
            pl.BlockSpec(memory_space=pl.ANY),
        ],
        out_shape=[
            jax.ShapeDtypeStruct(embeddings.shape, embeddings.dtype),
            jax.ShapeDtypeStruct(blade_masks.shape, blade_masks.dtype),
        ],
        scratch_shapes=[pltpu.SemaphoreType.DMA((_N_CHUNKS,)),
                        pltpu.SemaphoreType.DMA],
    )(embeddings, blade_masks)
    return (emb_out, masks_out)


# speedup vs baseline: 28.9873x; 28.9873x over previous
import jax
from jax.experimental import pallas as pl
from jax.experimental.pallas import tpu as pltpu

_ROWS = 100000
_GRID = 50
_RBLK = _ROWS // _GRID
_MBLK = _ROWS // _GRID


def _copy_body(emb_in, masks_in, emb_out, masks_out):
    emb_out[...] = emb_in[...]
    masks_out[...] = masks_in[...]


def kernel(embeddings, blade_masks):
    emb_out, masks_out = pl.pallas_call(
        _copy_body,
        grid=(_GRID,),
        in_specs=[
            pl.BlockSpec((_RBLK, 128), lambda i: (i, 0)),
            pl.BlockSpec((8, _ROWS), lambda i: (0, 0)),
        ],
        out_specs=[
            pl.BlockSpec((_RBLK, 128), lambda i: (i, 0)),
            pl.BlockSpec((8, _ROWS), lambda i: (0, 0)),
        ],
        out_shape=[
            jax.ShapeDtypeStruct(embeddings.shape, embeddings.dtype),
            jax.ShapeDtypeStruct(blade_masks.shape, blade_masks.dtype),
        ],
    )(embeddings, blade_masks)
    return (emb_out, masks_out)


# speedup vs baseline: 42.3247x; 1.4601x over previous
import jax
from jax.experimental import pallas as pl
from jax.experimental.pallas import tpu as pltpu

_ROWS = 100000
_GRID = 20
_RBLK = _ROWS // _GRID
_MBLK = _ROWS // _GRID


def _copy_body(emb_in, masks_in, emb_out, masks_out):
    emb_out[...] = emb_in[...]
    masks_out[...] = masks_in[...]


def kernel(embeddings, blade_masks):
    emb_out, masks_out = pl.pallas_call(
        _copy_body,
        grid=(_GRID,),
        in_specs=[
            pl.BlockSpec((_RBLK, 128), lambda i: (i, 0)),
            pl.BlockSpec((8, _ROWS), lambda i: (0, 0)),
        ],
        out_specs=[
            pl.BlockSpec((_RBLK, 128), lambda i: (i, 0)),
            pl.BlockSpec((8, _ROWS), lambda i: (0, 0)),
        ],
        out_shape=[
            jax.ShapeDtypeStruct(embeddings.shape, embeddings.dtype),
            jax.ShapeDtypeStruct(blade_masks.shape, blade_masks.dtype),
        ],
    )(embeddings, blade_masks)
    return (emb_out, masks_out)


# speedup vs baseline: 46.4227x; 1.0968x over previous
import jax
from jax.experimental import pallas as pl
from jax.experimental.pallas import tpu as pltpu

_ROWS = 100000
_GRID = 10
_RBLK = _ROWS // _GRID
_MBLK = _ROWS // _GRID


def _copy_body(emb_in, masks_in, emb_out, masks_out):
    emb_out[...] = emb_in[...]
    masks_out[...] = masks_in[...]


def kernel(embeddings, blade_masks):
    emb_out, masks_out = pl.pallas_call(
        _copy_body,
        grid=(_GRID,),
        in_specs=[
            pl.BlockSpec((_RBLK, 128), lambda i: (i, 0)),
            pl.BlockSpec((8, _ROWS), lambda i: (0, 0)),
        ],
        out_specs=[
            pl.BlockSpec((_RBLK, 128), lambda i: (i, 0)),
            pl.BlockSpec((8, _ROWS), lambda i: (0, 0)),
        ],
        out_shape=[
            jax.ShapeDtypeStruct(embeddings.shape, embeddings.dtype),
            jax.ShapeDtypeStruct(blade_masks.shape, blade_masks.dtype),
        ],
    )(embeddings, blade_masks)
    return (emb_out, masks_out)


# speedup vs baseline: 48.2653x; 1.0397x over previous
import jax
from jax.experimental import pallas as pl
from jax.experimental.pallas import tpu as pltpu

_ROWS = 100000
_GRID = 5
_RBLK = _ROWS // _GRID
_MBLK = _ROWS // _GRID


def _copy_body(emb_in, masks_in, emb_out, masks_out):
    emb_out[...] = emb_in[...]
    masks_out[...] = masks_in[...]


def kernel(embeddings, blade_masks):
    emb_out, masks_out = pl.pallas_call(
        _copy_body,
        grid=(_GRID,),
        in_specs=[
            pl.BlockSpec((_RBLK, 128), lambda i: (i, 0)),
            pl.BlockSpec((8, _ROWS), lambda i: (0, 0)),
        ],
        out_specs=[
            pl.BlockSpec((_RBLK, 128), lambda i: (i, 0)),
            pl.BlockSpec((8, _ROWS), lambda i: (0, 0)),
        ],
        out_shape=[
            jax.ShapeDtypeStruct(embeddings.shape, embeddings.dtype),
            jax.ShapeDtypeStruct(blade_masks.shape, blade_masks.dtype),
        ],
    )(embeddings, blade_masks)
    return (emb_out, masks_out)
